# baseline (speedup 1.0000x reference)
import jax
import jax.numpy as jnp
from jax import lax
from jax.experimental import pallas as pl
from jax.experimental.pallas import tpu as pltpu
from jax.experimental.pallas import tpu_sc as plsc

N_NODES = 10000
N_EDGES = 320000
D_FEAT = 128
D_EDGE = 16
HIDDEN = 64

NC = 2
NS = 16
FG = D_EDGE // NC
BATCH = 128
NB = N_EDGES // BATCH
CHUNK_B = 16
N_CHUNKS = 10
N_PAD = 10240

_sc_mesh = plsc.VectorSubcoreMesh(core_axis_name="c", subcore_axis_name="s",
                                  num_cores=NC, num_subcores=NS)


NROW = FG * N_PAD // BATCH
RPT = NROW // NS


def _segment_sum_body(attr_hbm, idx_hbm, out_hbm, idx_c, vals_c, table, ident,
                      zbuf, acc_sh, sems):
    cid = lax.axis_index("c")
    sid = lax.axis_index("s")

    iota16 = lax.iota(jnp.int32, 16)
    zeros16f = jnp.zeros((16,), jnp.float32)

    @plsc.parallel_loop(0, 5)
    def _ident(t):
        for j in range(8):
            ident[t, pl.ds(j * 16, 16)] = iota16 + (t * 128 + j * 16)

    @plsc.parallel_loop(0, RPT)
    def _zb(r):
        for j in range(8):
            zbuf[r, pl.ds(j * 16, 16)] = zeros16f

    pltpu.sync_copy(zbuf, acc_sh.at[pl.ds(sid * RPT, RPT)])

    nb = jnp.where(sid < 4, 157, 156)
    base = 156 * sid + jnp.minimum(sid, 4)

    def cb0_of(k):
        return jnp.minimum(base + k * CHUNK_B, NB - CHUNK_B)

    def issue(k, b):
        cb0 = cb0_of(k)
        pltpu.async_copy(idx_hbm.at[pl.ds(cb0, CHUNK_B), 0], idx_c.at[b],
                         sems.at[b])
        pltpu.async_copy(attr_hbm.at[cid, pl.ds(cb0, CHUNK_B)], vals_c.at[b],
                         sems.at[b])

    for b in range(2):
        issue(b, b)

    @plsc.parallel_loop(0, NROW)
    def _zero(kz):
        for j in range(8):
            table[kz, pl.ds(j * 16, 16)] = zeros16f

    @pl.loop(0, N_CHUNKS // 2)
    def _ring(kk):
        for b in range(2):
            k = kk * 2 + b
            cb0 = cb0_of(k)
            pltpu.make_async_copy(idx_hbm.at[pl.ds(cb0, CHUNK_B), 0],
                                  idx_c.at[b], sems.at[b]).wait()
            pltpu.make_async_copy(attr_hbm.at[cid, pl.ds(cb0, CHUNK_B)],
                                  vals_c.at[b], sems.at[b]).wait()
            d0 = base + k * CHUNK_B - cb0
            vb = jnp.minimum(nb - k * CHUNK_B, CHUNK_B)

            @plsc.parallel_loop(0, vb, unroll=2)
            def _batch(bi):
                off = d0 + bi
                for j2 in range(BATCH // 16):
                    idxv = idx_c[b, off, pl.ds(j2 * 16, 16)]
                    row0 = lax.shift_right_logical(idxv, 7)
                    colv = lax.bitwise_and(idxv, 127)
                    for f in range(FG):
                        valv = vals_c[b, off, f, pl.ds(j2 * 16, 16)]
                        plsc.addupdate_scatter(
                            table, [row0 + (f * (N_PAD // BATCH)), colv], valv)

            @pl.when(k + 2 < N_CHUNKS)
            def _():
                issue(k + 2, b)

    plsc.subcore_barrier()
    for t in range(NROW // BATCH):
        pltpu.sync_copy(table.at[pl.ds(t * BATCH, BATCH)],
                        acc_sh.at[ident.at[t]], add=True)
    plsc.subcore_barrier()
    pltpu.sync_copy(acc_sh.at[pl.ds(sid * RPT, RPT)],
                    out_hbm.at[cid, pl.ds(sid * RPT, RPT)])


_segment_sum_sc = pl.kernel(
    _segment_sum_body,
    out_type=jax.ShapeDtypeStruct((NC, NROW, BATCH), jnp.float32),
    mesh=_sc_mesh,
    scratch_types=[
        pltpu.VMEM((2, CHUNK_B, BATCH), jnp.int32),
        pltpu.VMEM((2, CHUNK_B, FG, BATCH), jnp.float32),
        pltpu.VMEM((NROW, BATCH), jnp.float32),
        pltpu.VMEM((NROW // BATCH, BATCH), jnp.int32),
        pltpu.VMEM((RPT, BATCH), jnp.float32),
        pltpu.VMEM_SHARED((NROW, BATCH), jnp.float32),
        pltpu.SemaphoreType.DMA((2,)),
    ],
    compiler_params=pltpu.CompilerParams(use_tc_tiling_on_sc=False,
                                         needs_layout_passes=False),
)


ROW_BLK = 2048
LB = ROW_BLK // BATCH


def _xw_body(x_ref, w1a_ref, b1_ref, t_ref):
    t_ref[...] = jnp.dot(x_ref[...], w1a_ref[...],
                         preferred_element_type=jnp.float32) + b1_ref[...]


def _xw_tc(x, W1a, b1):
    grid = (pl.cdiv(N_NODES, ROW_BLK),)
    return pl.pallas_call(
        _xw_body,
        grid=grid,
        in_specs=[
            pl.BlockSpec((ROW_BLK, D_FEAT), lambda i: (i, 0)),
            pl.BlockSpec((D_FEAT, HIDDEN), lambda i: (0, 0)),
            pl.BlockSpec((1, HIDDEN), lambda i: (0, 0)),
        ],
        out_specs=pl.BlockSpec((ROW_BLK, HIDDEN), lambda i: (i, 0)),
        out_shape=jax.ShapeDtypeStruct((N_NODES, HIDDEN), jnp.float32),
    )(x, W1a, b1)


def _mlp_body(t_ref, p_ref, w1b0_ref, w1b1_ref, w2_ref,
              b2_ref, w3_ref, b3col_ref, o_ref):
    agg0 = jnp.concatenate([p_ref[0, :, a, :] for a in range(LB)], axis=1)
    agg1 = jnp.concatenate([p_ref[1, :, a, :] for a in range(LB)], axis=1)
    dn = (((0,), (0,)), ((), ()))
    h = t_ref[...]
    h = h + lax.dot_general(agg0, w1b0_ref[...], dn,
                            preferred_element_type=jnp.float32)
    h = h + lax.dot_general(agg1, w1b1_ref[...], dn,
                            preferred_element_type=jnp.float32)
    h = jnp.where(h >= 0, h, 0.01 * h)
    h = jnp.dot(h, w2_ref[...], preferred_element_type=jnp.float32) + b2_ref[...]
    h = jnp.where(h >= 0, h, 0.01 * h)
    oT = lax.dot_general(w3_ref[...], h, (((0,), (1,)), ((), ())),
                         preferred_element_type=jnp.float32)
    o_ref[...] = oT + b3col_ref[...]


def _node_mlp_tc(t1, partials5, W1b0, W1b1, W2, b2, W3, b3col):
    grid = (pl.cdiv(N_NODES, ROW_BLK),)
    return pl.pallas_call(
        _mlp_body,
        grid=grid,
        in_specs=[
            pl.BlockSpec((ROW_BLK, HIDDEN), lambda i: (i, 0)),
            pl.BlockSpec((NC, FG, LB, BATCH), lambda i: (0, 0, i, 0)),
            pl.BlockSpec((FG, HIDDEN), lambda i: (0, 0)),
            pl.BlockSpec((FG, HIDDEN), lambda i: (0, 0)),
            pl.BlockSpec((HIDDEN, HIDDEN), lambda i: (0, 0)),
            pl.BlockSpec((1, HIDDEN), lambda i: (0, 0)),
            pl.BlockSpec((HIDDEN, HIDDEN), lambda i: (0, 0)),
            pl.BlockSpec((HIDDEN, 1), lambda i: (0, 0)),
        ],
        out_specs=pl.BlockSpec((HIDDEN, ROW_BLK), lambda i: (0, i)),
        out_shape=jax.ShapeDtypeStruct((HIDDEN, N_NODES), jnp.float32),
    )(t1, partials5, W1b0, W1b1, W2, b2, W3, b3col)


def kernel(x, edge_index, edge_attr, u, batch, W1, b1, W2, b2, W3, b3):
    attr4 = edge_attr.T.reshape(NC, FG, NB, BATCH).transpose(0, 2, 1, 3)
    idx3 = edge_index.astype(jnp.int32).reshape(2, NB, BATCH).transpose(1, 0, 2)

    partials5 = _segment_sum_sc(attr4, idx3).reshape(
        NC, FG, N_PAD // BATCH, BATCH)

    t1 = _xw_tc(x, W1[:D_FEAT], b1.reshape(1, HIDDEN))

    outT = _node_mlp_tc(
        t1, partials5,
        W1[D_FEAT:D_FEAT + FG], W1[D_FEAT + FG:],
        W2, b2.reshape(1, HIDDEN),
        W3, b3.reshape(HIDDEN, 1),
    )
    return outT.T


# speedup vs baseline: 1.0371x; 1.0371x over previous
import jax
import jax.numpy as jnp
from jax import lax
from jax.experimental import pallas as pl
from jax.experimental.pallas import tpu as pltpu
from jax.experimental.pallas import tpu_sc as plsc

N_NODES = 10000
N_EDGES = 320000
D_FEAT = 128
D_EDGE = 16
HIDDEN = 64

NC = 2
NS = 16
FG = D_EDGE // NC
BATCH = 128
NB = N_EDGES // BATCH
CHUNK_B = 16
N_CHUNKS = 10
N_PAD = 10240

_sc_mesh = plsc.VectorSubcoreMesh(core_axis_name="c", subcore_axis_name="s",
                                  num_cores=NC, num_subcores=NS)


NROW = FG * N_PAD // BATCH
RPT = NROW // NS


def _segment_sum_body(attr_hbm, idx_hbm, out_hbm, idx_c, vals_c, table, ident,
                      zbuf, acc_sh, sems):
    cid = lax.axis_index("c")
    sid = lax.axis_index("s")

    iota16 = lax.iota(jnp.int32, 16)
    zeros16f = jnp.zeros((16,), jnp.float32)

    @plsc.parallel_loop(0, 5)
    def _ident(t):
        for j in range(8):
            ident[t, pl.ds(j * 16, 16)] = iota16 + (t * 128 + j * 16)

    @plsc.parallel_loop(0, RPT)
    def _zb(r):
        for j in range(8):
            zbuf[r, pl.ds(j * 16, 16)] = zeros16f

    pltpu.sync_copy(zbuf, acc_sh.at[pl.ds(sid * RPT, RPT)])

    nb = jnp.where(sid < 4, 157, 156)
    base = 156 * sid + jnp.minimum(sid, 4)

    def cb0_of(k):
        return jnp.minimum(base + k * CHUNK_B, NB - CHUNK_B)

    def issue(k, b):
        cb0 = cb0_of(k)
        pltpu.async_copy(idx_hbm.at[pl.ds(cb0, CHUNK_B), 0], idx_c.at[b],
                         sems.at[b])
        pltpu.async_copy(attr_hbm.at[cid, pl.ds(cb0, CHUNK_B)], vals_c.at[b],
                         sems.at[b])

    for b in range(2):
        issue(b, b)

    @plsc.parallel_loop(0, NROW)
    def _zero(kz):
        for j in range(8):
            table[kz, pl.ds(j * 16, 16)] = zeros16f

    @pl.loop(0, N_CHUNKS // 2)
    def _ring(kk):
        for b in range(2):
            k = kk * 2 + b
            cb0 = cb0_of(k)
            pltpu.make_async_copy(idx_hbm.at[pl.ds(cb0, CHUNK_B), 0],
                                  idx_c.at[b], sems.at[b]).wait()
            pltpu.make_async_copy(attr_hbm.at[cid, pl.ds(cb0, CHUNK_B)],
                                  vals_c.at[b], sems.at[b]).wait()
            d0 = base + k * CHUNK_B - cb0
            vb = jnp.minimum(nb - k * CHUNK_B, CHUNK_B)

            @plsc.parallel_loop(0, vb)
            def _batch(bi):
                off = d0 + bi
                for j2 in range(BATCH // 16):
                    idxv = idx_c[b, off, pl.ds(j2 * 16, 16)]
                    row0 = lax.shift_right_logical(idxv, 7)
                    colv = lax.bitwise_and(idxv, 127)
                    for f in range(FG):
                        valv = vals_c[b, off, f, pl.ds(j2 * 16, 16)]
                        plsc.addupdate_scatter(
                            table, [row0 + (f * (N_PAD // BATCH)), colv], valv)

            @pl.when(k + 2 < N_CHUNKS)
            def _():
                issue(k + 2, b)

    plsc.subcore_barrier()
    for t in range(NROW // BATCH):
        pltpu.sync_copy(table.at[pl.ds(t * BATCH, BATCH)],
                        acc_sh.at[ident.at[t]], add=True)
    plsc.subcore_barrier()
    pltpu.sync_copy(acc_sh.at[pl.ds(sid * RPT, RPT)],
                    out_hbm.at[cid, pl.ds(sid * RPT, RPT)])


_segment_sum_sc = pl.kernel(
    _segment_sum_body,
    out_type=jax.ShapeDtypeStruct((NC, NROW, BATCH), jnp.float32),
    mesh=_sc_mesh,
    scratch_types=[
        pltpu.VMEM((2, CHUNK_B, BATCH), jnp.int32),
        pltpu.VMEM((2, CHUNK_B, FG, BATCH), jnp.float32),
        pltpu.VMEM((NROW, BATCH), jnp.float32),
        pltpu.VMEM((NROW // BATCH, BATCH), jnp.int32),
        pltpu.VMEM((RPT, BATCH), jnp.float32),
        pltpu.VMEM_SHARED((NROW, BATCH), jnp.float32),
        pltpu.SemaphoreType.DMA((2,)),
    ],
    compiler_params=pltpu.CompilerParams(use_tc_tiling_on_sc=False,
                                         needs_layout_passes=False),
)


ROW_BLK = 2048
LB = ROW_BLK // BATCH


def _xw_body(x_ref, w1a_ref, b1_ref, t_ref):
    t_ref[...] = jnp.dot(x_ref[...], w1a_ref[...],
                         preferred_element_type=jnp.float32) + b1_ref[...]


def _xw_tc(x, W1a, b1):
    grid = (pl.cdiv(N_NODES, ROW_BLK),)
    return pl.pallas_call(
        _xw_body,
        grid=grid,
        in_specs=[
            pl.BlockSpec((ROW_BLK, D_FEAT), lambda i: (i, 0)),
            pl.BlockSpec((D_FEAT, HIDDEN), lambda i: (0, 0)),
            pl.BlockSpec((1, HIDDEN), lambda i: (0, 0)),
        ],
        out_specs=pl.BlockSpec((ROW_BLK, HIDDEN), lambda i: (i, 0)),
        out_shape=jax.ShapeDtypeStruct((N_NODES, HIDDEN), jnp.float32),
    )(x, W1a, b1)


def _mlp_body(t_ref, p_ref, w1b0_ref, w1b1_ref, w2_ref,
              b2_ref, w3_ref, b3col_ref, o_ref):
    agg0 = jnp.concatenate([p_ref[0, :, a, :] for a in range(LB)], axis=1)
    agg1 = jnp.concatenate([p_ref[1, :, a, :] for a in range(LB)], axis=1)
    dn = (((0,), (0,)), ((), ()))
    h = t_ref[...]
    h = h + lax.dot_general(agg0, w1b0_ref[...], dn,
                            preferred_element_type=jnp.float32)
    h = h + lax.dot_general(agg1, w1b1_ref[...], dn,
                            preferred_element_type=jnp.float32)
    h = jnp.where(h >= 0, h, 0.01 * h)
    h = jnp.dot(h, w2_ref[...], preferred_element_type=jnp.float32) + b2_ref[...]
    h = jnp.where(h >= 0, h, 0.01 * h)
    oT = lax.dot_general(w3_ref[...], h, (((0,), (1,)), ((), ())),
                         preferred_element_type=jnp.float32)
    o_ref[...] = oT + b3col_ref[...]


def _node_mlp_tc(t1, partials5, W1b0, W1b1, W2, b2, W3, b3col):
    grid = (pl.cdiv(N_NODES, ROW_BLK),)
    return pl.pallas_call(
        _mlp_body,
        grid=grid,
        in_specs=[
            pl.BlockSpec((ROW_BLK, HIDDEN), lambda i: (i, 0)),
            pl.BlockSpec((NC, FG, LB, BATCH), lambda i: (0, 0, i, 0)),
            pl.BlockSpec((FG, HIDDEN), lambda i: (0, 0)),
            pl.BlockSpec((FG, HIDDEN), lambda i: (0, 0)),
            pl.BlockSpec((HIDDEN, HIDDEN), lambda i: (0, 0)),
            pl.BlockSpec((1, HIDDEN), lambda i: (0, 0)),
            pl.BlockSpec((HIDDEN, HIDDEN), lambda i: (0, 0)),
            pl.BlockSpec((HIDDEN, 1), lambda i: (0, 0)),
        ],
        out_specs=pl.BlockSpec((HIDDEN, ROW_BLK), lambda i: (0, i)),
        out_shape=jax.ShapeDtypeStruct((HIDDEN, N_NODES), jnp.float32),
    )(t1, partials5, W1b0, W1b1, W2, b2, W3, b3col)


def kernel(x, edge_index, edge_attr, u, batch, W1, b1, W2, b2, W3, b3):
    attr4 = edge_attr.T.reshape(NC, FG, NB, BATCH).transpose(0, 2, 1, 3)
    idx3 = edge_index.astype(jnp.int32).reshape(2, NB, BATCH).transpose(1, 0, 2)

    partials5 = _segment_sum_sc(attr4, idx3).reshape(
        NC, FG, N_PAD // BATCH, BATCH)

    t1 = _xw_tc(x, W1[:D_FEAT], b1.reshape(1, HIDDEN))

    outT = _node_mlp_tc(
        t1, partials5,
        W1[D_FEAT:D_FEAT + FG], W1[D_FEAT + FG:],
        W2, b2.reshape(1, HIDDEN),
        W3, b3.reshape(HIDDEN, 1),
    )
    return outT.T
